# baseline (speedup 1.0000x reference)
import jax
import jax.numpy as jnp
from jax.experimental import pallas as pl

_N = 10000
_M = 2000


def _gcn_conv(x, edge_index, W, b):
    n = x.shape[0]
    xw = x @ W
    src, dst = edge_index[0], edge_index[1]
    loop = jnp.arange(n, dtype=src.dtype)
    src = jnp.concatenate([src, loop])
    dst = jnp.concatenate([dst, loop])
    deg = jnp.zeros((n,), xw.dtype).at[dst].add(1.0)
    dinv = jax.lax.rsqrt(jnp.maximum(deg, 1.0))
    norm = dinv[src] * dinv[dst]
    msg = jnp.take(xw, src, axis=0) * norm[:, None]
    out = jnp.zeros((n, xw.shape[1]), xw.dtype).at[dst].add(msg)
    return out + b


def _hypergraph_conv(x, hyperedge_index, W, b, num_hyperedges):
    n = x.shape[0]
    xw = x @ W
    node, he = hyperedge_index[0], hyperedge_index[1]
    Dn = jnp.zeros((n,), xw.dtype).at[node].add(1.0)
    Be = jnp.zeros((num_hyperedges,), xw.dtype).at[he].add(1.0)
    m = jnp.zeros((num_hyperedges, xw.shape[1]), xw.dtype).at[he].add(jnp.take(xw, node, axis=0))
    m = m / jnp.maximum(Be, 1.0)[:, None]
    out = jnp.zeros((n, xw.shape[1]), xw.dtype).at[node].add(jnp.take(m, he, axis=0))
    out = out / jnp.maximum(Dn, 1.0)[:, None]
    return out + b


def _attention(x, x_hyper):
    s = jnp.stack([x.sum(-1), x_hyper.sum(-1)], axis=0)
    a = jax.nn.softmax(s, axis=0)
    return a[0][:, None] * x + a[1][:, None] * x_hyper


def _finale_body(x2_ref, xh2_ref, out_ref):
    x2 = x2_ref[...]
    xh2 = xh2_ref[...]
    s0 = jnp.sum(x2, axis=-1, keepdims=True)
    s1 = jnp.sum(xh2, axis=-1, keepdims=True)
    mx = jnp.maximum(s0, s1)
    e0 = jnp.exp(s0 - mx)
    e1 = jnp.exp(s1 - mx)
    z = e0 + e1
    out = (e0 / z) * x2 + (e1 / z) * xh2
    omax = jnp.max(out, axis=-1, keepdims=True)
    lse = jnp.log(jnp.sum(jnp.exp(out - omax), axis=-1, keepdims=True)) + omax
    out_ref[...] = out - lse


def kernel(x, edge_index, hyperedge_index, W1, b1, Wh1, bh1, W2, b2, Wh2, bh2):
    x1 = jax.nn.relu(_gcn_conv(x, edge_index, W1, b1))
    xh = jax.nn.relu(_hypergraph_conv(x1, hyperedge_index, Wh1, bh1, _M))
    xa = _attention(x1, xh)
    x2 = _gcn_conv(xa, edge_index, W2, b2)
    xh2 = _hypergraph_conv(xh, hyperedge_index, Wh2, bh2, _M)
    C = x2.shape[1]
    out = pl.pallas_call(
        _finale_body,
        out_shape=jax.ShapeDtypeStruct((_N, C), jnp.float32),
        grid=(10,),
        in_specs=[pl.BlockSpec((_N // 10, C), lambda i: (i, 0)),
                  pl.BlockSpec((_N // 10, C), lambda i: (i, 0))],
        out_specs=pl.BlockSpec((_N // 10, C), lambda i: (i, 0)),
    )(x2, xh2)
    return out


# speedup vs baseline: 9.5710x; 9.5710x over previous
import functools

import jax
import jax.numpy as jnp
from jax import lax
from jax.experimental import pallas as pl
from jax.experimental.pallas import tpu as pltpu
from jax.experimental.pallas import tpu_sc as plsc

N = 10000
E = 320000
M = 2000
F_IN = 128
DIM = 64
C = 40

NP = 10240
MP = 2048
CP = 48

NC = 2
NS = 16
NW = NC * NS
K = 80

_mesh = lambda: plsc.VectorSubcoreMesh(core_axis_name="c", subcore_axis_name="s")


def _make_agg(n_tab, n_out, d):
    e_per_w = E // NW
    n_chunks = e_per_w // K
    tab_stripe = n_tab // NS
    out_stripe = n_out // NS

    @functools.partial(
        pl.kernel,
        out_type=jax.ShapeDtypeStruct((NC * n_out, d), jnp.float32),
        mesh=_mesh(),
        scratch_types=[
            pltpu.VMEM_SHARED((n_tab, d), jnp.float32),
            pltpu.VMEM_SHARED((n_out, d), jnp.float32),
            pltpu.VMEM((K,), jnp.int32),
            pltpu.VMEM((K,), jnp.int32),
            pltpu.VMEM((K, d), jnp.float32),
        ],
    )
    def agg(tab_hbm, src_hbm, dst_hbm, init0_hbm, init1_hbm, out_hbm,
            tab_sp, acc_sp, idx_s, idx_d, rows):
        c = lax.axis_index("c")
        s = lax.axis_index("s")
        wid = s * NC + c

        pltpu.sync_copy(tab_hbm.at[pl.ds(s * tab_stripe, tab_stripe)],
                        tab_sp.at[pl.ds(s * tab_stripe, tab_stripe)])

        @pl.when(c == 0)
        def _():
            pltpu.sync_copy(init0_hbm.at[pl.ds(s * out_stripe, out_stripe)],
                            acc_sp.at[pl.ds(s * out_stripe, out_stripe)])

        @pl.when(c != 0)
        def _():
            pltpu.sync_copy(init1_hbm.at[pl.ds(s * out_stripe, out_stripe)],
                            acc_sp.at[pl.ds(s * out_stripe, out_stripe)])

        plsc.subcore_barrier()

        base = wid * e_per_w

        def body(i, carry):
            off = base + i * K
            pltpu.sync_copy(src_hbm.at[pl.ds(off, K)], idx_s)
            pltpu.sync_copy(dst_hbm.at[pl.ds(off, K)], idx_d)
            pltpu.sync_copy(tab_sp.at[idx_s], rows)
            pltpu.sync_copy(rows, acc_sp.at[idx_d], add=True)
            return carry

        lax.fori_loop(0, n_chunks, body, 0)

        plsc.subcore_barrier()

        pltpu.sync_copy(acc_sp.at[pl.ds(s * out_stripe, out_stripe)],
                        out_hbm.at[pl.ds(c * n_out + s * out_stripe, out_stripe)])

    return agg


def _make_degrees():
    e_per_w = E // NW
    n_chunks = e_per_w // K
    stripe = NP // NS

    @functools.partial(
        pl.kernel,
        out_type=jax.ShapeDtypeStruct((NC * 3 * NP, 16), jnp.float32),
        mesh=_mesh(),
        scratch_types=[
            pltpu.VMEM_SHARED((NP, 16), jnp.float32),
            pltpu.VMEM_SHARED((NP, 16), jnp.float32),
            pltpu.VMEM_SHARED((NP, 16), jnp.float32),
            pltpu.VMEM((K,), jnp.int32),
            pltpu.VMEM((K, 16), jnp.float32),
        ],
    )
    def degrees(dst_hbm, node_hbm, he_hbm, ones_hbm, zeros_hbm, out_hbm,
                h_dst, h_node, h_he, idx, ones_rows):
        c = lax.axis_index("c")
        s = lax.axis_index("s")
        wid = s * NC + c

        pltpu.sync_copy(ones_hbm, ones_rows)
        for acc in (h_dst, h_node, h_he):
            pltpu.sync_copy(zeros_hbm.at[pl.ds(s * stripe, stripe)],
                            acc.at[pl.ds(s * stripe, stripe)])
        plsc.subcore_barrier()

        base = wid * e_per_w

        def body(i, carry):
            off = base + i * K
            for src_hbm, acc in ((dst_hbm, h_dst), (node_hbm, h_node),
                                 (he_hbm, h_he)):
                pltpu.sync_copy(src_hbm.at[pl.ds(off, K)], idx)
                pltpu.sync_copy(ones_rows, acc.at[idx], add=True)
            return carry

        lax.fori_loop(0, n_chunks, body, 0)

        plsc.subcore_barrier()

        for h, acc in enumerate((h_dst, h_node, h_he)):
            pltpu.sync_copy(
                acc.at[pl.ds(s * stripe, stripe)],
                out_hbm.at[pl.ds((c * 3 + h) * NP + s * stripe, stripe)])

    return degrees


_BLK = 1024
_GRID = NP // _BLK


def _tc_call(body, out_widths, in_arrays, in_widths):
    n_out = len(out_widths)
    in_specs = []
    for a, w in zip(in_arrays, in_widths):
        if w is None:
            in_specs.append(pl.BlockSpec(a.shape, lambda i, nd=a.ndim: (0,) * nd))
        else:
            in_specs.append(pl.BlockSpec((_BLK, w), lambda i: (i, 0)))
    return pl.pallas_call(
        body,
        grid=(_GRID,),
        in_specs=in_specs,
        out_specs=[pl.BlockSpec((_BLK, w), lambda i: (i, 0)) for w in out_widths],
        out_shape=[jax.ShapeDtypeStruct((NP, w), jnp.float32) for w in out_widths],
    )(*in_arrays)


def _xw_scale_body(x_ref, w_ref, dinv_ref, y_ref):
    xw = jnp.dot(x_ref[...], w_ref[...], preferred_element_type=jnp.float32)
    y_ref[...] = xw * dinv_ref[...]


def _gcn_post_body(a0_ref, a1_ref, dinv_ref, b_ref, w_ref, x1_ref, z1_ref):
    x1 = jnp.maximum((a0_ref[...] + a1_ref[...]) * dinv_ref[...] + b_ref[...], 0.0)
    x1_ref[...] = x1
    z1_ref[...] = jnp.dot(x1, w_ref[...], preferred_element_type=jnp.float32)


def _scale_body(m0_ref, m1_ref, beinv_ref, m_ref):
    m_ref[...] = (m0_ref[...] + m1_ref[...]) * beinv_ref[...]


def _attn(x, xh):
    s0 = jnp.sum(x, axis=-1, keepdims=True)
    s1 = jnp.sum(xh, axis=-1, keepdims=True)
    mx = jnp.maximum(s0, s1)
    e0 = jnp.exp(s0 - mx)
    e1 = jnp.exp(s1 - mx)
    z = e0 + e1
    return (e0 / z) * x + (e1 / z) * xh


def _mid_body(h0_ref, h1_ref, dninv_ref, bh_ref, x1_ref, dinv_ref,
              w2_ref, wh2_ref, y2_ref, z2_ref):
    xh = jnp.maximum((h0_ref[...] + h1_ref[...]) * dninv_ref[...] + bh_ref[...], 0.0)
    xa = _attn(x1_ref[...], xh)
    y2_ref[...] = jnp.dot(xa, w2_ref[...], preferred_element_type=jnp.float32) * dinv_ref[...]
    z2_ref[...] = jnp.dot(xh, wh2_ref[...], preferred_element_type=jnp.float32)


def _finale_body(a0_ref, a1_ref, dinv_ref, b2_ref, h0_ref, h1_ref,
                 dninv_ref, bh2_ref, out_ref):
    x2 = (a0_ref[...] + a1_ref[...]) * dinv_ref[...] + b2_ref[...]
    xh2 = (h0_ref[...] + h1_ref[...]) * dninv_ref[...] + bh2_ref[...]
    out = _attn(x2, xh2)
    lanes = lax.broadcasted_iota(jnp.int32, out.shape, 1)
    out = jnp.where(lanes < C, out, -jnp.inf)
    omax = jnp.max(out, axis=-1, keepdims=True)
    lse = jnp.log(jnp.sum(jnp.exp(out - omax), axis=-1, keepdims=True)) + omax
    out_ref[...] = out - lse


def kernel(x, edge_index, hyperedge_index, W1, b1, Wh1, bh1, W2, b2, Wh2, bh2):
    f32 = jnp.float32
    src = edge_index[0]
    dst = edge_index[1]
    node = hyperedge_index[0]
    he = hyperedge_index[1]

    x_pad = jnp.zeros((NP, F_IN), f32).at[:N].set(x)
    zeros_n64 = jnp.zeros((NP, DIM), f32)
    zeros_n48 = jnp.zeros((NP, CP), f32)
    zeros_m64 = jnp.zeros((MP, DIM), f32)
    zeros_m48 = jnp.zeros((MP, CP), f32)
    zeros_deg = jnp.zeros((NP, 16), f32)
    ones_rows = jnp.ones((K, 16), f32)

    W2p = jnp.zeros((DIM, CP), f32).at[:, :C].set(W2)
    Wh2p = jnp.zeros((DIM, CP), f32).at[:, :C].set(Wh2)
    b1r = jnp.reshape(b1, (1, DIM))
    bh1r = jnp.reshape(bh1, (1, DIM))
    b2p = jnp.zeros((1, CP), f32).at[0, :C].set(b2)
    bh2p = jnp.zeros((1, CP), f32).at[0, :C].set(bh2)

    degs = _make_degrees()(dst, node, he, ones_rows, zeros_deg)
    degs = degs.reshape(NC, 3, NP, 16)
    counts = degs[0, :, :, 0] + degs[1, :, :, 0]
    deg = counts[0] + 1.0
    dinv = lax.rsqrt(deg)[:, None]
    dninv = (1.0 / jnp.maximum(counts[1], 1.0))[:, None]
    beinv = (1.0 / jnp.maximum(counts[2][:MP], 1.0))[:, None]

    (y1,) = _tc_call(_xw_scale_body, [DIM],
                     [x_pad, W1, dinv], [F_IN, None, 1])

    agg1 = _make_agg(NP, NP, DIM)(y1, src, dst, y1, zeros_n64)
    a0, a1 = agg1[:NP], agg1[NP:]

    x1, z1 = _tc_call(_gcn_post_body, [DIM, DIM],
                      [a0, a1, dinv, b1r, Wh1], [DIM, DIM, 1, None, None])

    mr = _make_agg(NP, MP, DIM)(z1, node, he, zeros_m64, zeros_m64)
    m = pl.pallas_call(
        _scale_body,
        grid=(2,),
        in_specs=[pl.BlockSpec((MP // 2, DIM), lambda i: (i, 0)),
                  pl.BlockSpec((MP // 2, DIM), lambda i: (i, 0)),
                  pl.BlockSpec((MP // 2, 1), lambda i: (i, 0))],
        out_specs=pl.BlockSpec((MP // 2, DIM), lambda i: (i, 0)),
        out_shape=jax.ShapeDtypeStruct((MP, DIM), f32),
    )(mr[:MP], mr[MP:], beinv)

    hg = _make_agg(MP, NP, DIM)(m, he, node, zeros_n64, zeros_n64)

    y2, z2 = _tc_call(_mid_body, [CP, CP],
                      [hg[:NP], hg[NP:], dninv, bh1r, x1, dinv, W2p, Wh2p],
                      [DIM, DIM, 1, None, DIM, 1, None, None])

    agg2 = _make_agg(NP, NP, CP)(y2, src, dst, y2, zeros_n48)

    mr2 = _make_agg(NP, MP, CP)(z2, node, he, zeros_m48, zeros_m48)
    m2 = pl.pallas_call(
        _scale_body,
        grid=(2,),
        in_specs=[pl.BlockSpec((MP // 2, CP), lambda i: (i, 0)),
                  pl.BlockSpec((MP // 2, CP), lambda i: (i, 0)),
                  pl.BlockSpec((MP // 2, 1), lambda i: (i, 0))],
        out_specs=pl.BlockSpec((MP // 2, CP), lambda i: (i, 0)),
        out_shape=jax.ShapeDtypeStruct((MP, CP), f32),
    )(mr2[:MP], mr2[MP:], beinv)

    hg2 = _make_agg(MP, NP, CP)(m2, he, node, zeros_n48, zeros_n48)

    (out,) = _tc_call(_finale_body, [CP],
                      [agg2[:NP], agg2[NP:], dinv, b2p, hg2[:NP], hg2[NP:],
                       dninv, bh2p],
                      [CP, CP, 1, None, CP, CP, 1, None])

    return out[:N, :C]


# speedup vs baseline: 19.9545x; 2.0849x over previous
import functools

import jax
import jax.numpy as jnp
from jax import lax
from jax.experimental import pallas as pl
from jax.experimental.pallas import tpu as pltpu
from jax.experimental.pallas import tpu_sc as plsc

N = 10000
E = 320000
M = 2000
F_IN = 128
DIM = 64
C = 40

NP = 10240
MP = 2048
CP = 48

NC = 2
NS = 16
NW = NC * NS
K = 40
NSP = 10016

_mesh = lambda: plsc.VectorSubcoreMesh(core_axis_name="c", subcore_axis_name="s")
_SC_PARAMS = pltpu.CompilerParams(use_tc_tiling_on_sc=False)


G = 5
_NCH = (E // NW) // K
_NG = _NCH // G


def _make_agg(n_tab, n_acc, d, out_stride, banks=2, k=K):
    tab_stripe = n_tab // NS
    out_stripe = n_acc // NS
    nch = (E // NW) // k
    ng = nch // G

    @functools.partial(
        pl.kernel,
        out_type=jax.ShapeDtypeStruct((NC * out_stride, d), jnp.float32),
        mesh=_mesh(),
        scratch_types=[
            pltpu.VMEM_SHARED((n_tab, d), jnp.float32),
            pltpu.VMEM_SHARED((n_acc, d), jnp.float32),
            pltpu.VMEM((nch, k), jnp.int32),
            pltpu.VMEM((nch, k), jnp.int32),
            pltpu.VMEM((banks * G, k, d), jnp.float32),
            pltpu.SemaphoreType.DMA,
            pltpu.SemaphoreType.DMA,
            pltpu.SemaphoreType.DMA,
            pltpu.SemaphoreType.DMA,
        ],
        compiler_params=_SC_PARAMS,
    )
    def agg(tab_hbm, src_hbm, dst_hbm, init0_hbm, init1_hbm, out_hbm,
            tab_sp, acc_sp, sidx, didx, rows, gsem0, gsem1, ssem0, ssem1):
        c = lax.axis_index("c")
        s = lax.axis_index("s")
        wid = s * NC + c
        gsem = (gsem0, gsem1)
        ssem = (ssem0, ssem1)

        pltpu.sync_copy(src_hbm.at[wid], sidx)
        pltpu.sync_copy(dst_hbm.at[wid], didx)
        pltpu.sync_copy(tab_hbm.at[pl.ds(s * tab_stripe, tab_stripe)],
                        tab_sp.at[pl.ds(s * tab_stripe, tab_stripe)])

        @pl.when(c == 0)
        def _():
            pltpu.sync_copy(init0_hbm.at[pl.ds(s * out_stripe, out_stripe)],
                            acc_sp.at[pl.ds(s * out_stripe, out_stripe)])

        @pl.when(c != 0)
        def _():
            pltpu.sync_copy(init1_hbm.at[pl.ds(s * out_stripe, out_stripe)],
                            acc_sp.at[pl.ds(s * out_stripe, out_stripe)])

        plsc.subcore_barrier()

        def gathers(g, bank):
            for b in range(G):
                pltpu.async_copy(tab_sp.at[sidx.at[g * G + b]],
                                 rows.at[bank * G + b], gsem[bank])

        def scatters(g, bank):
            for b in range(G):
                pltpu.async_copy(rows.at[bank * G + b],
                                 acc_sp.at[didx.at[g * G + b]],
                                 ssem[bank], add=True)

        def drain(sem_pair, bank):
            for b in range(G):
                pltpu.make_async_copy(tab_hbm.at[pl.ds(0, k)],
                                      rows.at[bank * G + b],
                                      sem_pair[bank]).wait()

        if banks == 2:
            def step(g, p, q):
                @pl.when(g < ng)
                def _():
                    drain(gsem, p)

                    @pl.when(g >= 1)
                    def _():
                        drain(ssem, q)

                    @pl.when(g + 1 < ng)
                    def _():
                        gathers(g + 1, q)

                    scatters(g, p)

            gathers(0, 0)

            def body(i, carry):
                step(2 * i, 0, 1)
                step(2 * i + 1, 1, 0)
                return carry

            lax.fori_loop(0, (ng + 1) // 2, body, 0)
            drain(ssem, (ng - 1) % 2)
        else:
            def body(g, carry):
                gathers(g, 0)
                drain(gsem, 0)
                scatters(g, 0)
                drain(ssem, 0)
                return carry

            lax.fori_loop(0, ng, body, 0)

        plsc.subcore_barrier()

        pltpu.sync_copy(
            acc_sp.at[pl.ds(s * out_stripe, out_stripe)],
            out_hbm.at[pl.ds(c * out_stride + s * out_stripe, out_stripe)])

    return agg


def _make_degrees():
    e_per_w = E // NW
    n_chunks = e_per_w // K
    stripe = NP // NS

    @functools.partial(
        pl.kernel,
        out_type=jax.ShapeDtypeStruct((NC * 3 * NP, 16), jnp.float32),
        mesh=_mesh(),
        scratch_types=[
            pltpu.VMEM_SHARED((NP, 16), jnp.float32),
            pltpu.VMEM_SHARED((NP, 16), jnp.float32),
            pltpu.VMEM_SHARED((NP, 16), jnp.float32),
            pltpu.VMEM((_NCH, K), jnp.int32),
            pltpu.VMEM((_NCH, K), jnp.int32),
            pltpu.VMEM((_NCH, K), jnp.int32),
            pltpu.VMEM((K, 16), jnp.float32),
            pltpu.SemaphoreType.DMA,
            pltpu.SemaphoreType.DMA,
        ],
        compiler_params=_SC_PARAMS,
    )
    def degrees(dst_hbm, node_hbm, he_hbm, ones_hbm, zeros_hbm, out_hbm,
                h_dst, h_node, h_he, i_dst, i_node, i_he, ones_rows,
                ssem0, ssem1):
        c = lax.axis_index("c")
        s = lax.axis_index("s")
        wid = s * NC + c
        ssem = (ssem0, ssem1)

        pltpu.sync_copy(ones_hbm, ones_rows)
        pltpu.sync_copy(dst_hbm.at[wid], i_dst)
        pltpu.sync_copy(node_hbm.at[wid], i_node)
        pltpu.sync_copy(he_hbm.at[wid], i_he)
        for acc in (h_dst, h_node, h_he):
            pltpu.sync_copy(zeros_hbm.at[pl.ds(s * stripe, stripe)],
                            acc.at[pl.ds(s * stripe, stripe)])
        plsc.subcore_barrier()

        def scatters(g, bank):
            for b in range(G):
                for idx3, acc in ((i_dst, h_dst), (i_node, h_node),
                                  (i_he, h_he)):
                    pltpu.async_copy(ones_rows, acc.at[idx3.at[g * G + b]],
                                     ssem[bank], add=True)

        def drain(bank):
            for _ in range(3 * G):
                pltpu.make_async_copy(ones_hbm, ones_rows, ssem[bank]).wait()

        def step(g, p, q):
            @pl.when(g < _NG)
            def _():
                @pl.when(g >= 1)
                def _():
                    drain(q)

                scatters(g, p)

        def body(i, carry):
            step(2 * i, 0, 1)
            step(2 * i + 1, 1, 0)
            return carry

        lax.fori_loop(0, (_NG + 1) // 2, body, 0)
        drain((_NG - 1) % 2)

        plsc.subcore_barrier()

        for h, acc in enumerate((h_dst, h_node, h_he)):
            pltpu.sync_copy(
                acc.at[pl.ds(s * stripe, stripe)],
                out_hbm.at[pl.ds((c * 3 + h) * NP + s * stripe, stripe)])

    return degrees


_BLK = 1024
_GRID = NP // _BLK


def _tc_call(body, out_widths, in_arrays, in_widths):
    n_out = len(out_widths)
    in_specs = []
    for a, w in zip(in_arrays, in_widths):
        if w is None:
            in_specs.append(pl.BlockSpec(a.shape, lambda i, nd=a.ndim: (0,) * nd))
        else:
            in_specs.append(pl.BlockSpec((_BLK, w), lambda i: (i, 0)))
    return pl.pallas_call(
        body,
        grid=(_GRID,),
        in_specs=in_specs,
        out_specs=[pl.BlockSpec((_BLK, w), lambda i: (i, 0)) for w in out_widths],
        out_shape=[jax.ShapeDtypeStruct((NP, w), jnp.float32) for w in out_widths],
    )(*in_arrays)


def _xw_scale_body(x_ref, w_ref, dinv_ref, y_ref):
    xw = jnp.dot(x_ref[...], w_ref[...], preferred_element_type=jnp.float32)
    y_ref[...] = xw * dinv_ref[...]


def _gcn_post_body(a0a_ref, a1a_ref, a0b_ref, a1b_ref, dinv_ref, b_ref,
                   w_ref, x1_ref, z1_ref):
    agg = jnp.concatenate([a0a_ref[...] + a1a_ref[...],
                           a0b_ref[...] + a1b_ref[...]], axis=1)
    x1 = jnp.maximum(agg * dinv_ref[...] + b_ref[...], 0.0)
    x1_ref[...] = x1
    z1_ref[...] = jnp.dot(x1, w_ref[...], preferred_element_type=jnp.float32)


def _scale_body(m0_ref, m1_ref, beinv_ref, m_ref):
    m_ref[...] = (m0_ref[...] + m1_ref[...]) * beinv_ref[...]


def _attn(x, xh):
    s0 = jnp.sum(x, axis=-1, keepdims=True)
    s1 = jnp.sum(xh, axis=-1, keepdims=True)
    mx = jnp.maximum(s0, s1)
    e0 = jnp.exp(s0 - mx)
    e1 = jnp.exp(s1 - mx)
    z = e0 + e1
    return (e0 / z) * x + (e1 / z) * xh


def _mid_body(h0_ref, h1_ref, dninv_ref, bh_ref, x1_ref, dinv_ref,
              w2_ref, wh2_ref, y2_ref, z2_ref):
    xh = jnp.maximum((h0_ref[...] + h1_ref[...]) * dninv_ref[...] + bh_ref[...], 0.0)
    xa = _attn(x1_ref[...], xh)
    y2_ref[...] = jnp.dot(xa, w2_ref[...], preferred_element_type=jnp.float32) * dinv_ref[...]
    z2_ref[...] = jnp.dot(xh, wh2_ref[...], preferred_element_type=jnp.float32)


def _finale_body(a0_ref, a1_ref, dinv_ref, b2_ref, h0_ref, h1_ref,
                 dninv_ref, bh2_ref, out_ref):
    x2 = (a0_ref[...] + a1_ref[...]) * dinv_ref[...] + b2_ref[...]
    xh2 = (h0_ref[...] + h1_ref[...]) * dninv_ref[...] + bh2_ref[...]
    out = _attn(x2, xh2)
    lanes = lax.broadcasted_iota(jnp.int32, out.shape, 1)
    out = jnp.where(lanes < C, out, -jnp.inf)
    omax = jnp.max(out, axis=-1, keepdims=True)
    lse = jnp.log(jnp.sum(jnp.exp(out - omax), axis=-1, keepdims=True)) + omax
    out_ref[...] = out - lse


def kernel(x, edge_index, hyperedge_index, W1, b1, Wh1, bh1, W2, b2, Wh2, bh2):
    f32 = jnp.float32
    src = edge_index[0].reshape(NW, _NCH, K)
    dst = edge_index[1].reshape(NW, _NCH, K)
    node = hyperedge_index[0].reshape(NW, _NCH, K)
    he = hyperedge_index[1].reshape(NW, _NCH, K)

    x_pad = jnp.zeros((NP, F_IN), f32).at[:N].set(x)
    zeros_n64 = jnp.zeros((NP, DIM), f32)
    zeros_n48 = jnp.zeros((NP, CP), f32)
    zeros_m64 = jnp.zeros((MP, DIM), f32)
    zeros_m48 = jnp.zeros((MP, CP), f32)
    zeros_deg = jnp.zeros((NP, 16), f32)
    ones_rows = jnp.ones((K, 16), f32)

    W2p = jnp.zeros((DIM, CP), f32).at[:, :C].set(W2)
    Wh2p = jnp.zeros((DIM, CP), f32).at[:, :C].set(Wh2)
    b1r = jnp.reshape(b1, (1, DIM))
    bh1r = jnp.reshape(bh1, (1, DIM))
    b2p = jnp.zeros((1, CP), f32).at[0, :C].set(b2)
    bh2p = jnp.zeros((1, CP), f32).at[0, :C].set(bh2)

    degs = _make_degrees()(dst, node, he, ones_rows, zeros_deg)
    degs = degs.reshape(NC, 3, NP, 16)
    counts = degs[0, :, :, 0] + degs[1, :, :, 0]
    deg = counts[0] + 1.0
    dinv = lax.rsqrt(deg)[:, None]
    dninv = (1.0 / jnp.maximum(counts[1], 1.0))[:, None]
    beinv = (1.0 / jnp.maximum(counts[2][:MP], 1.0))[:, None]

    (y1,) = _tc_call(_xw_scale_body, [DIM],
                     [x_pad, W1, dinv], [F_IN, None, 1])

    y1a, y1b = y1[:, :32], y1[:, 32:]
    zeros_n32 = jnp.zeros((NP, 32), f32)
    agg1a = _make_agg(NSP, NSP, 32, NP)(y1a, src, dst, y1a, zeros_n32)
    agg1b = _make_agg(NSP, NSP, 32, NP)(y1b, src, dst, y1b, zeros_n32)

    x1, z1 = _tc_call(_gcn_post_body, [DIM, DIM],
                      [agg1a[:NP], agg1a[NP:], agg1b[:NP], agg1b[NP:],
                       dinv, b1r, Wh1], [32, 32, 32, 32, 1, None, None])

    mr = _make_agg(NSP, MP, DIM, MP)(z1, node, he, zeros_m64, zeros_m64)
    m = pl.pallas_call(
        _scale_body,
        grid=(2,),
        in_specs=[pl.BlockSpec((MP // 2, DIM), lambda i: (i, 0)),
                  pl.BlockSpec((MP // 2, DIM), lambda i: (i, 0)),
                  pl.BlockSpec((MP // 2, 1), lambda i: (i, 0))],
        out_specs=pl.BlockSpec((MP // 2, DIM), lambda i: (i, 0)),
        out_shape=jax.ShapeDtypeStruct((MP, DIM), f32),
    )(mr[:MP], mr[MP:], beinv)

    hg = _make_agg(MP, NSP, DIM, NP)(m, he, node, zeros_n64, zeros_n64)

    y2, z2 = _tc_call(_mid_body, [CP, CP],
                      [hg[:NP], hg[NP:], dninv, bh1r, x1, dinv, W2p, Wh2p],
                      [DIM, DIM, 1, None, DIM, 1, None, None])

    agg2 = _make_agg(NSP, NSP, CP, NP, banks=1)(y2, src, dst, y2, zeros_n48)

    mr2 = _make_agg(NSP, MP, CP, MP)(z2, node, he, zeros_m48, zeros_m48)
    m2 = pl.pallas_call(
        _scale_body,
        grid=(2,),
        in_specs=[pl.BlockSpec((MP // 2, CP), lambda i: (i, 0)),
                  pl.BlockSpec((MP // 2, CP), lambda i: (i, 0)),
                  pl.BlockSpec((MP // 2, 1), lambda i: (i, 0))],
        out_specs=pl.BlockSpec((MP // 2, CP), lambda i: (i, 0)),
        out_shape=jax.ShapeDtypeStruct((MP, CP), f32),
    )(mr2[:MP], mr2[MP:], beinv)

    hg2 = _make_agg(MP, NSP, CP, NP)(m2, he, node, zeros_n48, zeros_n48)

    (out,) = _tc_call(_finale_body, [CP],
                      [agg2[:NP], agg2[NP:], dinv, b2p, hg2[:NP], hg2[NP:],
                       dninv, bh2p],
                      [CP, CP, 1, None, CP, CP, 1, None])

    return out[:N, :C]


# speedup vs baseline: 21.3633x; 1.0706x over previous
import functools

import jax
import jax.numpy as jnp
from jax import lax
from jax.experimental import pallas as pl
from jax.experimental.pallas import tpu as pltpu
from jax.experimental.pallas import tpu_sc as plsc

N = 10000
E = 320000
M = 2000
F_IN = 128
DIM = 64
C = 40

NP = 10240
NSP = 10016
MP = 2048
CP = 48

NC = 2
NS = 16
NW = NC * NS
EPW = 10240
E_PAD = NW * EPW

_mesh = lambda: plsc.VectorSubcoreMesh(core_axis_name="c", subcore_axis_name="s")
_SC_PARAMS = pltpu.CompilerParams(use_tc_tiling_on_sc=False)


def _make_agg(n_tab, n_acc, d, out_stride, k=128, g=2):
    tab_stripe = n_tab // NS
    out_stripe = n_acc // NS
    nch = EPW // k
    ng = nch // g

    @functools.partial(
        pl.kernel,
        out_type=jax.ShapeDtypeStruct((NC * out_stride, d), jnp.float32),
        mesh=_mesh(),
        scratch_types=[
            pltpu.VMEM_SHARED((n_tab, d), jnp.float32),
            pltpu.VMEM_SHARED((n_acc, d), jnp.float32),
            pltpu.VMEM((nch, k), jnp.int32),
            pltpu.VMEM((nch, k), jnp.int32),
            pltpu.VMEM((2 * g, k, d), jnp.float32),
            pltpu.SemaphoreType.DMA,
            pltpu.SemaphoreType.DMA,
            pltpu.SemaphoreType.DMA,
            pltpu.SemaphoreType.DMA,
        ],
        compiler_params=_SC_PARAMS,
    )
    def agg(tab_hbm, src_hbm, dst_hbm, init0_hbm, init1_hbm, out_hbm,
            tab_sp, acc_sp, sidx, didx, rows, gsem0, gsem1, ssem0, ssem1):
        c = lax.axis_index("c")
        s = lax.axis_index("s")
        wid = s * NC + c
        gsem = (gsem0, gsem1)
        ssem = (ssem0, ssem1)

        pltpu.sync_copy(src_hbm.at[wid], sidx)
        pltpu.sync_copy(dst_hbm.at[wid], didx)
        pltpu.sync_copy(tab_hbm.at[pl.ds(s * tab_stripe, tab_stripe)],
                        tab_sp.at[pl.ds(s * tab_stripe, tab_stripe)])

        @pl.when(c == 0)
        def _():
            pltpu.sync_copy(init0_hbm.at[pl.ds(s * out_stripe, out_stripe)],
                            acc_sp.at[pl.ds(s * out_stripe, out_stripe)])

        @pl.when(c != 0)
        def _():
            pltpu.sync_copy(init1_hbm.at[pl.ds(s * out_stripe, out_stripe)],
                            acc_sp.at[pl.ds(s * out_stripe, out_stripe)])

        plsc.subcore_barrier()

        def gathers(grp, bank):
            for b in range(g):
                pltpu.async_copy(tab_sp.at[sidx.at[grp * g + b]],
                                 rows.at[bank * g + b], gsem[bank])

        def scatters(grp, bank):
            for b in range(g):
                pltpu.async_copy(rows.at[bank * g + b],
                                 acc_sp.at[didx.at[grp * g + b]],
                                 ssem[bank], add=True)

        def drain(sem_pair, bank):
            for b in range(g):
                pltpu.make_async_copy(tab_hbm.at[pl.ds(0, k)],
                                      rows.at[bank * g + b],
                                      sem_pair[bank]).wait()

        def step(grp, p, q):
            @pl.when(grp < ng)
            def _():
                drain(gsem, p)

                @pl.when(grp >= 1)
                def _():
                    drain(ssem, q)

                @pl.when(grp + 1 < ng)
                def _():
                    gathers(grp + 1, q)

                scatters(grp, p)

        gathers(0, 0)

        def body(i, carry):
            step(2 * i, 0, 1)
            step(2 * i + 1, 1, 0)
            return carry

        lax.fori_loop(0, (ng + 1) // 2, body, 0)
        drain(ssem, (ng - 1) % 2)

        plsc.subcore_barrier()

        pltpu.sync_copy(
            acc_sp.at[pl.ds(s * out_stripe, out_stripe)],
            out_hbm.at[pl.ds(c * out_stride + s * out_stripe, out_stripe)])

    return agg


def _make_degrees(k=128, g=2):
    stripe = NP // NS
    nch = EPW // k
    ng = nch // g

    @functools.partial(
        pl.kernel,
        out_type=jax.ShapeDtypeStruct((NC * 3 * NP, 16), jnp.float32),
        mesh=_mesh(),
        scratch_types=[
            pltpu.VMEM_SHARED((NP, 16), jnp.float32),
            pltpu.VMEM_SHARED((NP, 16), jnp.float32),
            pltpu.VMEM_SHARED((NP, 16), jnp.float32),
            pltpu.VMEM((nch, k), jnp.int32),
            pltpu.VMEM((nch, k), jnp.int32),
            pltpu.VMEM((nch, k), jnp.int32),
            pltpu.VMEM((k, 16), jnp.float32),
            pltpu.SemaphoreType.DMA,
            pltpu.SemaphoreType.DMA,
        ],
        compiler_params=_SC_PARAMS,
    )
    def degrees(dst_hbm, node_hbm, he_hbm, ones_hbm, zeros_hbm, out_hbm,
                h_dst, h_node, h_he, i_dst, i_node, i_he, ones_rows,
                ssem0, ssem1):
        c = lax.axis_index("c")
        s = lax.axis_index("s")
        wid = s * NC + c
        ssem = (ssem0, ssem1)

        pltpu.sync_copy(ones_hbm, ones_rows)
        pltpu.sync_copy(dst_hbm.at[wid], i_dst)
        pltpu.sync_copy(node_hbm.at[wid], i_node)
        pltpu.sync_copy(he_hbm.at[wid], i_he)
        for acc in (h_dst, h_node, h_he):
            pltpu.sync_copy(zeros_hbm.at[pl.ds(s * stripe, stripe)],
                            acc.at[pl.ds(s * stripe, stripe)])
        plsc.subcore_barrier()

        def scatters(grp, bank):
            for b in range(g):
                for idx3, acc in ((i_dst, h_dst), (i_node, h_node),
                                  (i_he, h_he)):
                    pltpu.async_copy(ones_rows, acc.at[idx3.at[grp * g + b]],
                                     ssem[bank], add=True)

        def drain(bank):
            for _ in range(3 * g):
                pltpu.make_async_copy(ones_hbm, ones_rows, ssem[bank]).wait()

        def step(grp, p, q):
            @pl.when(grp < ng)
            def _():
                @pl.when(grp >= 1)
                def _():
                    drain(q)

                scatters(grp, p)

        def body(i, carry):
            step(2 * i, 0, 1)
            step(2 * i + 1, 1, 0)
            return carry

        lax.fori_loop(0, (ng + 1) // 2, body, 0)
        drain((ng - 1) % 2)

        plsc.subcore_barrier()

        for h, acc in enumerate((h_dst, h_node, h_he)):
            pltpu.sync_copy(
                acc.at[pl.ds(s * stripe, stripe)],
                out_hbm.at[pl.ds((c * 3 + h) * NP + s * stripe, stripe)])

    return degrees


_BLK = 1024
_GRID = NP // _BLK


def _tc_call(body, out_widths, in_arrays, in_widths):
    in_specs = []
    for a, w in zip(in_arrays, in_widths):
        if w is None:
            in_specs.append(pl.BlockSpec(a.shape, lambda i, nd=a.ndim: (0,) * nd))
        else:
            in_specs.append(pl.BlockSpec((_BLK, w), lambda i: (i, 0)))
    return pl.pallas_call(
        body,
        grid=(_GRID,),
        in_specs=in_specs,
        out_specs=[pl.BlockSpec((_BLK, w), lambda i: (i, 0)) for w in out_widths],
        out_shape=[jax.ShapeDtypeStruct((NP, w), jnp.float32) for w in out_widths],
    )(*in_arrays)


def _xw_scale_body(x_ref, w_ref, dinv_ref, y_ref):
    xw = jnp.dot(x_ref[...], w_ref[...], preferred_element_type=jnp.float32)
    y_ref[...] = xw * dinv_ref[...]


def _gcn_post_body(a0_ref, a1_ref, dinv_ref, b_ref, w_ref, x1_ref, z1_ref):
    x1 = jnp.maximum((a0_ref[...] + a1_ref[...]) * dinv_ref[...] + b_ref[...], 0.0)
    x1_ref[...] = x1
    z1_ref[...] = jnp.dot(x1, w_ref[...], preferred_element_type=jnp.float32)


def _scale_body(m0_ref, m1_ref, beinv_ref, m_ref):
    m_ref[...] = (m0_ref[...] + m1_ref[...]) * beinv_ref[...]


def _attn(x, xh):
    s0 = jnp.sum(x, axis=-1, keepdims=True)
    s1 = jnp.sum(xh, axis=-1, keepdims=True)
    mx = jnp.maximum(s0, s1)
    e0 = jnp.exp(s0 - mx)
    e1 = jnp.exp(s1 - mx)
    z = e0 + e1
    return (e0 / z) * x + (e1 / z) * xh


def _mid_body(h0_ref, h1_ref, dninv_ref, bh_ref, x1_ref, dinv_ref,
              w2_ref, wh2_ref, y2_ref, z2_ref):
    xh = jnp.maximum((h0_ref[...] + h1_ref[...]) * dninv_ref[...] + bh_ref[...], 0.0)
    xa = _attn(x1_ref[...], xh)
    y2_ref[...] = jnp.dot(xa, w2_ref[...], preferred_element_type=jnp.float32) * dinv_ref[...]
    z2_ref[...] = jnp.dot(xh, wh2_ref[...], preferred_element_type=jnp.float32)


def _finale_body(a0_ref, a1_ref, dinv_ref, b2_ref, h0_ref, h1_ref,
                 dninv_ref, bh2_ref, out_ref):
    x2 = (a0_ref[...] + a1_ref[...]) * dinv_ref[...] + b2_ref[...]
    xh2 = (h0_ref[...] + h1_ref[...]) * dninv_ref[...] + bh2_ref[...]
    out = _attn(x2, xh2)
    lanes = lax.broadcasted_iota(jnp.int32, out.shape, 1)
    out = jnp.where(lanes < C, out, -jnp.inf)
    omax = jnp.max(out, axis=-1, keepdims=True)
    lse = jnp.log(jnp.sum(jnp.exp(out - omax), axis=-1, keepdims=True)) + omax
    out_ref[...] = out - lse


def kernel(x, edge_index, hyperedge_index, W1, b1, Wh1, bh1, W2, b2, Wh2, bh2):
    f32 = jnp.float32
    i32 = jnp.int32

    fill = jnp.arange(E_PAD - E, dtype=i32) % 16
    srcp = jnp.concatenate([edge_index[0], N + fill])
    dstp = jnp.concatenate([edge_index[1], N + fill])
    nodep = jnp.concatenate([hyperedge_index[0], N + fill])
    hep = jnp.concatenate([hyperedge_index[1], M + fill])

    src = srcp.reshape(NW, EPW // 128, 128)
    dst = dstp.reshape(NW, EPW // 128, 128)
    node = nodep.reshape(NW, EPW // 128, 128)
    he = hep.reshape(NW, EPW // 128, 128)
    src80 = srcp.reshape(NW, EPW // 80, 80)
    dst80 = dstp.reshape(NW, EPW // 80, 80)

    x_pad = jnp.zeros((NP, F_IN), f32).at[:N].set(x)
    zeros_n64 = jnp.zeros((NP, DIM), f32)
    zeros_n48 = jnp.zeros((NP, CP), f32)
    zeros_m64 = jnp.zeros((MP, DIM), f32)
    zeros_m48 = jnp.zeros((MP, CP), f32)
    zeros_deg = jnp.zeros((NP, 16), f32)
    ones_rows = jnp.ones((128, 16), f32)

    W2p = jnp.zeros((DIM, CP), f32).at[:, :C].set(W2)
    Wh2p = jnp.zeros((DIM, CP), f32).at[:, :C].set(Wh2)
    b1r = jnp.reshape(b1, (1, DIM))
    bh1r = jnp.reshape(bh1, (1, DIM))
    b2p = jnp.zeros((1, CP), f32).at[0, :C].set(b2)
    bh2p = jnp.zeros((1, CP), f32).at[0, :C].set(bh2)

    degs = _make_degrees()(dst, node, he, ones_rows, zeros_deg)
    degs = degs.reshape(NC, 3, NP, 16)
    counts = degs[0, :, :, 0] + degs[1, :, :, 0]
    deg = counts[0] + 1.0
    dinv = lax.rsqrt(deg)[:, None]
    dninv = (1.0 / jnp.maximum(counts[1], 1.0))[:, None]
    beinv = (1.0 / jnp.maximum(counts[2][:MP], 1.0))[:, None]

    (y1,) = _tc_call(_xw_scale_body, [DIM],
                     [x_pad, W1, dinv], [F_IN, None, 1])

    agg1 = _make_agg(NSP, NSP, DIM, NP, k=80)(y1, src80, dst80, y1, zeros_n64)

    x1, z1 = _tc_call(_gcn_post_body, [DIM, DIM],
                      [agg1[:NP], agg1[NP:], dinv, b1r, Wh1],
                      [DIM, DIM, 1, None, None])

    mr = _make_agg(NSP, MP, DIM, MP)(z1, node, he, zeros_m64, zeros_m64)
    m = pl.pallas_call(
        _scale_body,
        grid=(2,),
        in_specs=[pl.BlockSpec((MP // 2, DIM), lambda i: (i, 0)),
                  pl.BlockSpec((MP // 2, DIM), lambda i: (i, 0)),
                  pl.BlockSpec((MP // 2, 1), lambda i: (i, 0))],
        out_specs=pl.BlockSpec((MP // 2, DIM), lambda i: (i, 0)),
        out_shape=jax.ShapeDtypeStruct((MP, DIM), f32),
    )(mr[:MP], mr[MP:], beinv)

    hg = _make_agg(MP, NSP, DIM, NP)(m, he, node, zeros_n64, zeros_n64)

    y2, z2 = _tc_call(_mid_body, [CP, CP],
                      [hg[:NP], hg[NP:], dninv, bh1r, x1, dinv, W2p, Wh2p],
                      [DIM, DIM, 1, None, DIM, 1, None, None])

    agg2 = _make_agg(NSP, NSP, CP, NP)(y2, src, dst, y2, zeros_n48)

    mr2 = _make_agg(NSP, MP, CP, MP)(z2, node, he, zeros_m48, zeros_m48)
    m2 = pl.pallas_call(
        _scale_body,
        grid=(2,),
        in_specs=[pl.BlockSpec((MP // 2, CP), lambda i: (i, 0)),
                  pl.BlockSpec((MP // 2, CP), lambda i: (i, 0)),
                  pl.BlockSpec((MP // 2, 1), lambda i: (i, 0))],
        out_specs=pl.BlockSpec((MP // 2, CP), lambda i: (i, 0)),
        out_shape=jax.ShapeDtypeStruct((MP, CP), f32),
    )(mr2[:MP], mr2[MP:], beinv)

    hg2 = _make_agg(MP, NSP, CP, NP)(m2, he, node, zeros_n48, zeros_n48)

    (out,) = _tc_call(_finale_body, [CP],
                      [agg2[:NP], agg2[NP:], dinv, b2p, hg2[:NP], hg2[NP:],
                       dninv, bh2p],
                      [CP, CP, 1, None, CP, CP, 1, None])

    return out[:N, :C]
